# Initial kernel scaffold; baseline (speedup 1.0000x reference)
import jax
import jax.numpy as jnp
from jax.experimental import pallas as pl


def kernel(past_lens, past_ids, category_id, created_at, words_count, age, hour_of_day, day_of_week, environment, deviceGroup, os, country, region, referrer_type, content_embedding, item_table, category_id_table, created_at_table, words_count_table, age_table, hour_of_day_table, day_of_week_table, environment_table, deviceGroup_table, os_table, country_table, region_table, referrer_type_table, pos_table, W, b):
    raise NotImplementedError("write your pallas kernel here")


# speedup vs baseline: 1.2838x; 1.2838x over previous
import functools

import jax
import jax.numpy as jnp
from jax import lax
from jax.experimental import pallas as pl
from jax.experimental.pallas import tpu as pltpu
from jax.experimental.pallas import tpu_sc as plsc

_NC = 2
_NS = 16
_NW = _NC * _NS
_K = 256


def _sc_gather_body(*refs):
    tables = refs[0:7]
    idxs = refs[7:14]
    atabs = refs[14:20]
    aidxs = refs[20:26]
    outs = refs[26:33]
    aouts = refs[33:39]
    idx_v, rows_v, aidx_v, arows_v, sem = refs[39:44]

    wid = lax.axis_index("s") * _NC + lax.axis_index("c")
    n_win = outs[0].shape[0] // (_NW * _K)
    base0 = wid * (n_win * _K)
    for t in range(7):
        @pl.loop(0, n_win)
        def _(i, t=t):
            base = base0 + i * _K
            pltpu.sync_copy(idxs[t].at[pl.ds(base, _K)], idx_v)
            pltpu.async_copy(tables[t].at[idx_v], rows_v, sem).wait()
            pltpu.sync_copy(rows_v, outs[t].at[pl.ds(base, _K)])

    a_per_w = aouts[0].shape[0] // _NW
    abase = wid * a_per_w
    for t in range(6):
        pltpu.sync_copy(aidxs[t].at[pl.ds(abase, a_per_w)], aidx_v)
        pltpu.async_copy(atabs[t].at[aidx_v], arows_v, sem).wait()
        pltpu.sync_copy(arows_v, aouts[t].at[pl.ds(abase, a_per_w)])


def _tc_body(gs0, gs1, gs2, gs3, gs4, gs5, gs6,
             aux0, aux1, aux2, aux3, aux4, aux5,
             content, pids, lens, pos, w, bias,
             seq_o, valid_o, mask_o, lens_o):
    bb, n, d = seq_o.shape
    scale = float(d) ** 0.5
    gs = (gs0[:, : n - 1] + gs1[:, : n - 1] + gs2[:, : n - 1]
          + gs3[:, : n - 1] + gs4[:, : n - 1] + gs5[:, : n - 1]
          + gs6[:, : n - 1])
    c = jnp.dot(content[...].reshape(bb * n, content.shape[2]), w[...],
                preferred_element_type=jnp.float32).reshape(bb, n, d)
    pos_v = pos[...]
    seqpart = (gs + c[:, : n - 1] + bias[...][None]) * scale + pos_v[None, 1:n]
    auxpart = (aux0[...] + aux1[...] + aux2[...] + aux3[...] + aux4[...]
               + aux5[...]) * scale + pos_v[0][None]
    validf = (pids[:, : n - 1] != 0).astype(jnp.float32)
    seq_o[...] = jnp.concatenate(
        [auxpart[:, None, :], seqpart * validf[..., None]], axis=1)
    valid_o[...] = jnp.concatenate(
        [jnp.ones((bb, 1), jnp.float32), validf], axis=1)
    lens1 = lens[...] + 1
    lens_o[...] = lens1
    mask_o[...] = (lax.broadcasted_iota(jnp.int32, (bb, n), 1)
                   < lens1).astype(jnp.int32)


def kernel(past_lens, past_ids, category_id, created_at, words_count, age, hour_of_day, day_of_week, environment, deviceGroup, os, country, region, referrer_type, content_embedding, item_table, category_id_table, created_at_table, words_count_table, age_table, hour_of_day_table, day_of_week_table, environment_table, deviceGroup_table, os_table, country_table, region_table, referrer_type_table, pos_table, W, b):
    B, N = past_ids.shape
    D = item_table.shape[1]
    P = B * N

    seq_tables = (item_table, category_id_table, created_at_table,
                  words_count_table, age_table, hour_of_day_table,
                  day_of_week_table)
    seq_idx = tuple(a.reshape(P) for a in
                    (past_ids, category_id, created_at, words_count, age,
                     hour_of_day, day_of_week))
    aux_tables = (environment_table, deviceGroup_table, os_table,
                  country_table, region_table, referrer_type_table)
    aux_idx = (environment, deviceGroup, os, country, region, referrer_type)

    mesh = plsc.VectorSubcoreMesh(core_axis_name="c", subcore_axis_name="s")
    sc_gather = functools.partial(
        pl.kernel, mesh=mesh,
        compiler_params=pltpu.CompilerParams(use_tc_tiling_on_sc=False),
        out_type=([jax.ShapeDtypeStruct((P, D), jnp.float32)] * 7
                  + [jax.ShapeDtypeStruct((B, D), jnp.float32)] * 6),
        scratch_types=[
            pltpu.VMEM((_K,), jnp.int32),
            pltpu.VMEM((_K, D), jnp.float32),
            pltpu.VMEM((B // _NW,), jnp.int32),
            pltpu.VMEM((B // _NW, D), jnp.float32),
            pltpu.SemaphoreType.DMA,
        ],
    )(_sc_gather_body)

    sc_out = sc_gather(*seq_tables, *seq_idx, *aux_tables, *aux_idx)
    gs = [o.reshape(B, N, D) for o in sc_out[:7]]
    aux = sc_out[7:13]

    BB = 16
    grid = (B // BB,)
    seq, valid, mask_i, lens_o = pl.pallas_call(
        _tc_body,
        grid=grid,
        in_specs=(
            [pl.BlockSpec((BB, N, D), lambda i: (i, 0, 0))] * 7
            + [pl.BlockSpec((BB, D), lambda i: (i, 0))] * 6
            + [pl.BlockSpec((BB, N, content_embedding.shape[2]),
                            lambda i: (i, 0, 0)),
               pl.BlockSpec((BB, N), lambda i: (i, 0)),
               pl.BlockSpec((BB, 1), lambda i: (i, 0)),
               pl.BlockSpec((N, D), lambda i: (0, 0)),
               pl.BlockSpec(W.shape, lambda i: (0, 0)),
               pl.BlockSpec((1, D), lambda i: (0, 0))]
        ),
        out_specs=[
            pl.BlockSpec((BB, N, D), lambda i: (i, 0, 0)),
            pl.BlockSpec((BB, N), lambda i: (i, 0)),
            pl.BlockSpec((BB, N), lambda i: (i, 0)),
            pl.BlockSpec((BB, 1), lambda i: (i, 0)),
        ],
        out_shape=[
            jax.ShapeDtypeStruct((B, N, D), jnp.float32),
            jax.ShapeDtypeStruct((B, N), jnp.float32),
            jax.ShapeDtypeStruct((B, N), jnp.int32),
            jax.ShapeDtypeStruct((B, 1), jnp.int32),
        ],
    )(*gs, *aux, content_embedding, past_ids, past_lens.reshape(B, 1),
      pos_table, W, b.reshape(1, D))

    return (lens_o.reshape(B), seq, valid[..., None], mask_i.astype(jnp.bool_))


# speedup vs baseline: 1.3018x; 1.0140x over previous
import functools

import jax
import jax.numpy as jnp
from jax import lax
from jax.experimental import pallas as pl
from jax.experimental.pallas import tpu as pltpu
from jax.experimental.pallas import tpu_sc as plsc

_NC = 2
_NS = 16
_NW = _NC * _NS
_K = 128


def _sc_gather_body(*refs):
    tables = refs[0:7]
    idxs = refs[7:14]
    atabs = refs[14:20]
    aidxs = refs[20:26]
    outs = refs[26:33]
    aouts = refs[33:39]
    idx_vs = refs[39:46]
    row_vs = refs[46:53]
    aidx_v, arows_v, sem, wsem = refs[53:57]

    wid = lax.axis_index("s") * _NC + lax.axis_index("c")
    per_w = outs[0].shape[0] // _NW
    n_win = per_w // _K
    base0 = wid * per_w

    cps = [pltpu.async_copy(idxs[t].at[pl.ds(base0, per_w)], idx_vs[t], sem)
           for t in range(7)]
    for cp in cps:
        cp.wait()

    @pl.loop(0, n_win)
    def _(i):
        off = i * _K
        gcps = [pltpu.async_copy(
            tables[t].at[idx_vs[t].at[pl.ds(off, _K)]], row_vs[t], sem)
            for t in range(7)]
        for cp in gcps:
            cp.wait()
        wcps = [pltpu.async_copy(
            row_vs[t], outs[t].at[pl.ds(base0 + off, _K)], wsem)
            for t in range(7)]
        for cp in wcps:
            cp.wait()

    a_per_w = aouts[0].shape[0] // _NW
    abase = wid * a_per_w
    for t in range(6):
        pltpu.sync_copy(aidxs[t].at[pl.ds(abase, a_per_w)], aidx_v)
        pltpu.async_copy(atabs[t].at[aidx_v], arows_v, sem).wait()
        pltpu.sync_copy(arows_v, aouts[t].at[pl.ds(abase, a_per_w)])


def _tc_body(gs0, gs1, gs2, gs3, gs4, gs5, gs6,
             aux0, aux1, aux2, aux3, aux4, aux5,
             content, pids, lens, pos, w, bias,
             seq_o, valid_o, mask_o, lens_o):
    bb, n, d = seq_o.shape
    scale = float(d) ** 0.5
    gs = (gs0[:, : n - 1] + gs1[:, : n - 1] + gs2[:, : n - 1]
          + gs3[:, : n - 1] + gs4[:, : n - 1] + gs5[:, : n - 1]
          + gs6[:, : n - 1])
    c = jnp.dot(content[...].reshape(bb * n, content.shape[2]), w[...],
                preferred_element_type=jnp.float32).reshape(bb, n, d)
    pos_v = pos[...]
    seqpart = (gs + c[:, : n - 1] + bias[...][None]) * scale + pos_v[None, 1:n]
    auxpart = (aux0[...] + aux1[...] + aux2[...] + aux3[...] + aux4[...]
               + aux5[...]) * scale + pos_v[0][None]
    validf = (pids[:, : n - 1] != 0).astype(jnp.float32)
    seq_o[...] = jnp.concatenate(
        [auxpart[:, None, :], seqpart * validf[..., None]], axis=1)
    valid_o[...] = jnp.concatenate(
        [jnp.ones((bb, 1), jnp.float32), validf], axis=1)[..., None]
    lens1 = lens[...] + 1
    lens_o[...] = lens1
    mask_o[...] = lax.broadcasted_iota(jnp.int32, (bb, n), 1) < lens1


def kernel(past_lens, past_ids, category_id, created_at, words_count, age, hour_of_day, day_of_week, environment, deviceGroup, os, country, region, referrer_type, content_embedding, item_table, category_id_table, created_at_table, words_count_table, age_table, hour_of_day_table, day_of_week_table, environment_table, deviceGroup_table, os_table, country_table, region_table, referrer_type_table, pos_table, W, b):
    B, N = past_ids.shape
    D = item_table.shape[1]
    P = B * N
    per_w = P // _NW

    seq_tables = (item_table, category_id_table, created_at_table,
                  words_count_table, age_table, hour_of_day_table,
                  day_of_week_table)
    seq_idx = tuple(a.reshape(P) for a in
                    (past_ids, category_id, created_at, words_count, age,
                     hour_of_day, day_of_week))
    aux_tables = (environment_table, deviceGroup_table, os_table,
                  country_table, region_table, referrer_type_table)
    aux_idx = (environment, deviceGroup, os, country, region, referrer_type)

    mesh = plsc.VectorSubcoreMesh(core_axis_name="c", subcore_axis_name="s")
    sc_gather = functools.partial(
        pl.kernel, mesh=mesh,
        compiler_params=pltpu.CompilerParams(use_tc_tiling_on_sc=False),
        out_type=([jax.ShapeDtypeStruct((P, D), jnp.float32)] * 7
                  + [jax.ShapeDtypeStruct((B, D), jnp.float32)] * 6),
        scratch_types=(
            [pltpu.VMEM((per_w,), jnp.int32)] * 7
            + [pltpu.VMEM((_K, D), jnp.float32)] * 7
            + [pltpu.VMEM((B // _NW,), jnp.int32),
               pltpu.VMEM((B // _NW, D), jnp.float32),
               pltpu.SemaphoreType.DMA,
               pltpu.SemaphoreType.DMA]
        ),
    )(_sc_gather_body)

    sc_out = sc_gather(*seq_tables, *seq_idx, *aux_tables, *aux_idx)
    gs = [o.reshape(B, N, D) for o in sc_out[:7]]
    aux = sc_out[7:13]

    BB = 16
    grid = (B // BB,)
    seq, valid, mask, lens_o = pl.pallas_call(
        _tc_body,
        grid=grid,
        in_specs=(
            [pl.BlockSpec((BB, N, D), lambda i: (i, 0, 0))] * 7
            + [pl.BlockSpec((BB, D), lambda i: (i, 0))] * 6
            + [pl.BlockSpec((BB, N, content_embedding.shape[2]),
                            lambda i: (i, 0, 0)),
               pl.BlockSpec((BB, N), lambda i: (i, 0)),
               pl.BlockSpec((BB, 1), lambda i: (i, 0)),
               pl.BlockSpec((N, D), lambda i: (0, 0)),
               pl.BlockSpec(W.shape, lambda i: (0, 0)),
               pl.BlockSpec((1, D), lambda i: (0, 0))]
        ),
        out_specs=[
            pl.BlockSpec((BB, N, D), lambda i: (i, 0, 0)),
            pl.BlockSpec((BB, N, 1), lambda i: (i, 0, 0)),
            pl.BlockSpec((BB, N), lambda i: (i, 0)),
            pl.BlockSpec((BB, 1), lambda i: (i, 0)),
        ],
        out_shape=[
            jax.ShapeDtypeStruct((B, N, D), jnp.float32),
            jax.ShapeDtypeStruct((B, N, 1), jnp.float32),
            jax.ShapeDtypeStruct((B, N), jnp.bool_),
            jax.ShapeDtypeStruct((B, 1), jnp.int32),
        ],
    )(*gs, *aux, content_embedding, past_ids, past_lens.reshape(B, 1),
      pos_table, W, b.reshape(1, D))

    return (lens_o.reshape(B), seq, valid, mask)


# speedup vs baseline: 1.8215x; 1.3993x over previous
import functools

import jax
import jax.numpy as jnp
from jax import lax
from jax.experimental import pallas as pl
from jax.experimental.pallas import tpu as pltpu
from jax.experimental.pallas import tpu_sc as plsc

_NC = 2
_NS = 16
_NW = _NC * _NS
_K = 128


def _sum_rows(row_vs, n_rows, d):
    n_bufs = len(row_vs)

    @pl.loop(0, n_rows)
    def _(r):
        for c in range(d // 16):
            slc = pl.ds(c * 16, 16)
            acc = row_vs[0][r, slc]
            for t in range(1, n_bufs):
                acc = acc + row_vs[t][r, slc]
            row_vs[0][r, slc] = acc


def _sc_gather_body(*refs):
    tables = refs[0:7]
    idxs = refs[7:14]
    atabs = refs[14:20]
    aidxs = refs[20:26]
    out = refs[26]
    aout = refs[27]
    idx_vs = refs[28:35]
    row_vs = refs[35:42]
    aidx_vs = refs[42:48]
    arow_vs = refs[48:54]
    sem, wsem = refs[54:56]

    wid = lax.axis_index("s") * _NC + lax.axis_index("c")
    per_w = out.shape[0] // _NW
    n_win = per_w // _K
    base0 = wid * per_w
    d = out.shape[1]

    cps = [pltpu.async_copy(idxs[t].at[pl.ds(base0, per_w)], idx_vs[t], sem)
           for t in range(7)]
    for cp in cps:
        cp.wait()

    @pl.loop(0, n_win)
    def _(i):
        off = i * _K
        gcps = [pltpu.async_copy(
            tables[t].at[idx_vs[t].at[pl.ds(off, _K)]], row_vs[t], sem)
            for t in range(7)]
        for cp in gcps:
            cp.wait()
        _sum_rows(row_vs, _K, d)
        pltpu.async_copy(row_vs[0], out.at[pl.ds(base0 + off, _K)],
                         wsem).wait()

    a_per_w = aout.shape[0] // _NW
    abase = wid * a_per_w
    acps = []
    for t in range(6):
        pltpu.sync_copy(aidxs[t].at[pl.ds(abase, a_per_w)], aidx_vs[t])
        acps.append(pltpu.async_copy(atabs[t].at[aidx_vs[t]], arow_vs[t], sem))
    for cp in acps:
        cp.wait()
    _sum_rows(arow_vs, a_per_w, d)
    pltpu.sync_copy(arow_vs[0], aout.at[pl.ds(abase, a_per_w)])


def _tc_body(gs0, aux0, content, pids, lens, pos, w, bias,
             seq_o, valid_o, mask_o, lens_o):
    bb, n, d = seq_o.shape
    scale = float(d) ** 0.5
    gs = gs0[:, : n - 1]
    c = jnp.dot(content[...].reshape(bb * n, content.shape[2]), w[...],
                preferred_element_type=jnp.float32).reshape(bb, n, d)
    pos_v = pos[...]
    seqpart = (gs + c[:, : n - 1] + bias[...][None]) * scale + pos_v[None, 1:n]
    auxpart = aux0[...] * scale + pos_v[0][None]
    validf = (pids[:, : n - 1] != 0).astype(jnp.float32)
    seq_o[...] = jnp.concatenate(
        [auxpart[:, None, :], seqpart * validf[..., None]], axis=1)
    valid_o[...] = jnp.concatenate(
        [jnp.ones((bb, 1), jnp.float32), validf], axis=1)[..., None]
    lens1 = lens[...] + 1
    lens_o[...] = lens1
    mask_o[...] = lax.broadcasted_iota(jnp.int32, (bb, n), 1) < lens1


def kernel(past_lens, past_ids, category_id, created_at, words_count, age, hour_of_day, day_of_week, environment, deviceGroup, os, country, region, referrer_type, content_embedding, item_table, category_id_table, created_at_table, words_count_table, age_table, hour_of_day_table, day_of_week_table, environment_table, deviceGroup_table, os_table, country_table, region_table, referrer_type_table, pos_table, W, b):
    B, N = past_ids.shape
    D = item_table.shape[1]
    P = B * N
    per_w = P // _NW

    seq_tables = (item_table, category_id_table, created_at_table,
                  words_count_table, age_table, hour_of_day_table,
                  day_of_week_table)
    seq_idx = tuple(a.reshape(P) for a in
                    (past_ids, category_id, created_at, words_count, age,
                     hour_of_day, day_of_week))
    aux_tables = (environment_table, deviceGroup_table, os_table,
                  country_table, region_table, referrer_type_table)
    aux_idx = (environment, deviceGroup, os, country, region, referrer_type)

    mesh = plsc.VectorSubcoreMesh(core_axis_name="c", subcore_axis_name="s")
    sc_gather = functools.partial(
        pl.kernel, mesh=mesh,
        compiler_params=pltpu.CompilerParams(use_tc_tiling_on_sc=False),
        out_type=[jax.ShapeDtypeStruct((P, D), jnp.float32),
                  jax.ShapeDtypeStruct((B, D), jnp.float32)],
        scratch_types=(
            [pltpu.VMEM((per_w,), jnp.int32)] * 7
            + [pltpu.VMEM((_K, D), jnp.float32)] * 7
            + [pltpu.VMEM((B // _NW,), jnp.int32)] * 6
            + [pltpu.VMEM((B // _NW, D), jnp.float32)] * 6
            + [pltpu.SemaphoreType.DMA,
               pltpu.SemaphoreType.DMA]
        ),
    )(_sc_gather_body)

    gs_flat, aux = sc_gather(*seq_tables, *seq_idx, *aux_tables, *aux_idx)
    gs = gs_flat.reshape(B, N, D)

    BB = 32
    grid = (B // BB,)
    seq, valid, mask, lens_o = pl.pallas_call(
        _tc_body,
        grid=grid,
        in_specs=(
            [pl.BlockSpec((BB, N, D), lambda i: (i, 0, 0)),
             pl.BlockSpec((BB, D), lambda i: (i, 0)),
             pl.BlockSpec((BB, N, content_embedding.shape[2]),
                          lambda i: (i, 0, 0)),
             pl.BlockSpec((BB, N), lambda i: (i, 0)),
             pl.BlockSpec((BB, 1), lambda i: (i, 0)),
             pl.BlockSpec((N, D), lambda i: (0, 0)),
             pl.BlockSpec(W.shape, lambda i: (0, 0)),
             pl.BlockSpec((1, D), lambda i: (0, 0))]
        ),
        out_specs=[
            pl.BlockSpec((BB, N, D), lambda i: (i, 0, 0)),
            pl.BlockSpec((BB, N, 1), lambda i: (i, 0, 0)),
            pl.BlockSpec((BB, N), lambda i: (i, 0)),
            pl.BlockSpec((BB, 1), lambda i: (i, 0)),
        ],
        out_shape=[
            jax.ShapeDtypeStruct((B, N, D), jnp.float32),
            jax.ShapeDtypeStruct((B, N, 1), jnp.float32),
            jax.ShapeDtypeStruct((B, N), jnp.bool_),
            jax.ShapeDtypeStruct((B, 1), jnp.int32),
        ],
    )(gs, aux, content_embedding, past_ids, past_lens.reshape(B, 1),
      pos_table, W, b.reshape(1, D))

    return (lens_o.reshape(B), seq, valid, mask)
